# baseline (speedup 1.0000x reference)
import functools

import numpy as np
import jax
import jax.numpy as jnp
from jax import lax
from jax.experimental import pallas as pl

_K = 64


def _knn_mask_kernel(pct_ref, tgt_ref, pnorm_ref, pcflat_ref, rexp_ref,
                     out_ref):
    pct = pct_ref[...]
    tgt = tgt_ref[...]
    dots = lax.dot_general(
        tgt, pct,
        dimension_numbers=(((2,), (1,)), ((0,), (0,))),
        preferred_element_type=jnp.float32,
    )
    np_ = pnorm_ref[...][:, None, :]
    nq = jnp.sum(tgt * tgt, axis=2, keepdims=True)
    d = (nq + np_) - 2.0 * dots
    d = jnp.maximum(d, 0.0)
    keys = lax.bitcast_convert_type(d, jnp.int32)

    g, q, _ = keys.shape

    prefix = jnp.zeros((g, q, 1), jnp.int32)
    for i in range(31):
        cand = prefix | jnp.int32(1 << (30 - i))
        cnt = jnp.sum(jnp.where(keys < cand, 1.0, 0.0), axis=2,
                      keepdims=True)
        prefix = jnp.where(cnt >= float(_K), prefix, cand)

    prefix_b = jnp.broadcast_to(prefix, keys.shape)
    mask = keys <= prefix_b
    mask_bf = jnp.where(mask, 1.0, 0.0).astype(jnp.bfloat16)
    rep = lax.dot_general(
        mask_bf, rexp_ref[...],
        dimension_numbers=(((2,), (0,)), ((), ())),
        preferred_element_type=jnp.float32,
    )
    out_ref[...] = rep * pcflat_ref[...][:, None, :]


@jax.jit
def kernel(pointcloud, target):
    pc = pointcloud[..., :3]
    b, n, _ = pc.shape
    q = target.shape[1]
    g = 64
    pct = jnp.transpose(pc, (0, 2, 1))
    pnorm = jnp.sum(pc * pc, axis=2)
    pcflat = pc.reshape(b, n * 3)
    rexp = jnp.asarray(np.repeat(np.eye(n, dtype=np.float32), 3, axis=1),
                       dtype=jnp.bfloat16)

    out = pl.pallas_call(
        _knn_mask_kernel,
        grid=(b // g,),
        in_specs=[
            pl.BlockSpec((g, 3, n), lambda i: (i, 0, 0)),
            pl.BlockSpec((g, q, 3), lambda i: (i, 0, 0)),
            pl.BlockSpec((g, n), lambda i: (i, 0)),
            pl.BlockSpec((g, n * 3), lambda i: (i, 0)),
            pl.BlockSpec((n, n * 3), lambda i: (0, 0)),
        ],
        out_specs=pl.BlockSpec((g, q, n * 3), lambda i: (i, 0, 0)),
        out_shape=jax.ShapeDtypeStruct((b, q, n * 3), jnp.float32),
    )(pct, target, pnorm, pcflat, rexp)
    return out.reshape(b, q, n, 3)


# speedup vs baseline: 1.1535x; 1.1535x over previous
import functools

import numpy as np
import jax
import jax.numpy as jnp
from jax import lax
from jax.experimental import pallas as pl

_K = 64


def _knn_mask_kernel(pct_ref, tgt_ref, pnorm_ref, pcflat_ref, rexp_ref,
                     out_ref):
    pct = pct_ref[...]
    tgt = tgt_ref[...]
    dots = lax.dot_general(
        tgt, pct,
        dimension_numbers=(((2,), (1,)), ((0,), (0,))),
        preferred_element_type=jnp.float32,
    )
    np_ = pnorm_ref[...][:, None, :]
    nq = jnp.sum(tgt * tgt, axis=2, keepdims=True)
    d = (nq + np_) - 2.0 * dots
    d = jnp.maximum(d, 0.0)
    keys = lax.bitcast_convert_type(d, jnp.int32)

    g, q, _ = keys.shape

    prefix = jnp.zeros((g, q, 1), jnp.int32)
    for i in range(31):
        cand = prefix | jnp.int32(1 << (30 - i))
        cnt = jnp.sum(jnp.where(keys < cand, 1.0, 0.0), axis=2,
                      keepdims=True)
        prefix = jnp.where(cnt >= float(_K), prefix, cand)

    prefix_b = jnp.broadcast_to(prefix, keys.shape)
    mask = keys <= prefix_b
    mask_bf = jnp.where(mask, 1.0, 0.0).astype(jnp.bfloat16)
    rep = lax.dot_general(
        mask_bf, rexp_ref[...],
        dimension_numbers=(((2,), (0,)), ((), ())),
        preferred_element_type=jnp.float32,
    )
    out_ref[...] = rep * pcflat_ref[...][:, None, :]


@jax.jit
def kernel(pointcloud, target):
    pc = pointcloud[..., :3]
    b, n, _ = pc.shape
    q = target.shape[1]
    g = 32
    pct = jnp.transpose(pc, (0, 2, 1))
    pnorm = jnp.sum(pc * pc, axis=2)
    pcflat = pc.reshape(b, n * 3)
    rexp = jnp.asarray(np.repeat(np.eye(n, dtype=np.float32), 3, axis=1),
                       dtype=jnp.bfloat16)

    out = pl.pallas_call(
        _knn_mask_kernel,
        grid=(b // g,),
        in_specs=[
            pl.BlockSpec((g, 3, n), lambda i: (i, 0, 0)),
            pl.BlockSpec((g, q, 3), lambda i: (i, 0, 0)),
            pl.BlockSpec((g, n), lambda i: (i, 0)),
            pl.BlockSpec((g, n * 3), lambda i: (i, 0)),
            pl.BlockSpec((n, n * 3), lambda i: (0, 0)),
        ],
        out_specs=pl.BlockSpec((g, q, n * 3), lambda i: (i, 0, 0)),
        out_shape=jax.ShapeDtypeStruct((b, q, n * 3), jnp.float32),
    )(pct, target, pnorm, pcflat, rexp)
    return out.reshape(b, q, n, 3)
